# baseline (speedup 1.0000x reference)
import jax
import jax.numpy as jnp
from jax.experimental import pallas as pl
from jax.experimental.pallas import tpu as pltpu

_N = 10000
_D = 128
_BM1 = 200
_BM2 = 400


def _transform_body(x_ref, w_ref, b_ref, o_ref):
    o_ref[...] = (
        jnp.dot(x_ref[...], w_ref[...], preferred_element_type=jnp.float32)
        + b_ref[...]
    )


def _spmm1_body(a_ref, g_ref, w_ref, b_ref, o_ref, acopy_ref, abf_ref):
    a = a_ref[...]
    acopy_ref[...] = a
    a_bf = a.astype(jnp.bfloat16)
    abf_ref[...] = a_bf
    y = jnp.dot(a_bf, g_ref[...].astype(jnp.bfloat16),
                preferred_element_type=jnp.float32)
    o_ref[...] = (
        jnp.dot(jnp.maximum(y, 0.0), w_ref[...], preferred_element_type=jnp.float32)
        + b_ref[...]
    )


def _spmm_mid_body(a_ref, g_ref, w_ref, b_ref, o_ref):
    y = jnp.dot(a_ref[...], g_ref[...].astype(jnp.bfloat16),
                preferred_element_type=jnp.float32)
    o_ref[...] = (
        jnp.dot(jnp.maximum(y, 0.0), w_ref[...], preferred_element_type=jnp.float32)
        + b_ref[...]
    )


def _spmm_last_body(a_ref, g_ref, o_ref):
    y = jnp.dot(a_ref[...], g_ref[...].astype(jnp.bfloat16),
                preferred_element_type=jnp.float32)
    denom = jnp.clip(jnp.sum(jnp.abs(y), axis=1, keepdims=True), 1e-12, None)
    o_ref[...] = y / denom


def kernel(x, A, W1, b1, W2, b2, W3, b3):
    f32 = jnp.float32
    g1 = pl.pallas_call(
        _transform_body,
        out_shape=jax.ShapeDtypeStruct((_N, _D), f32),
    )(x, W1, b1[None, :])

    params = pltpu.CompilerParams(dimension_semantics=("arbitrary",))

    def specs(bm):
        return dict(
            a=pl.BlockSpec((bm, _N), lambda i: (i, 0)),
            g=pl.BlockSpec((_N, _D), lambda i: (0, 0)),
            w=pl.BlockSpec((_D, _D), lambda i: (0, 0)),
            b=pl.BlockSpec((1, _D), lambda i: (0, 0)),
            o=pl.BlockSpec((bm, _D), lambda i: (i, 0)),
        )

    s1 = specs(_BM1)
    g2, A_copy, A_bf = pl.pallas_call(
        _spmm1_body,
        grid=(_N // _BM1,),
        in_specs=[s1["a"], s1["g"], s1["w"], s1["b"]],
        out_specs=[s1["o"], s1["a"], s1["a"]],
        out_shape=[
            jax.ShapeDtypeStruct((_N, _D), f32),
            jax.ShapeDtypeStruct((_N, _N), f32),
            jax.ShapeDtypeStruct((_N, _N), jnp.bfloat16),
        ],
        compiler_params=params,
    )(A, g1, W2, b2[None, :])

    s2 = specs(_BM2)
    g3 = pl.pallas_call(
        _spmm_mid_body,
        grid=(_N // _BM2,),
        in_specs=[s2["a"], s2["g"], s2["w"], s2["b"]],
        out_specs=s2["o"],
        out_shape=jax.ShapeDtypeStruct((_N, _D), f32),
        compiler_params=params,
    )(A_bf, g2, W3, b3[None, :])

    h = pl.pallas_call(
        _spmm_last_body,
        grid=(_N // _BM2,),
        in_specs=[s2["a"], s2["g"]],
        out_specs=s2["o"],
        out_shape=jax.ShapeDtypeStruct((_N, _D), f32),
        compiler_params=params,
    )(A_bf, g3)
    return (h, h, A_copy)


# speedup vs baseline: 1.0629x; 1.0629x over previous
import jax
import jax.numpy as jnp
from jax.experimental import pallas as pl
from jax.experimental.pallas import tpu as pltpu

_N = 10000
_D = 128
_BM1 = 200
_BM2 = 1000


def _transform_body(x_ref, w_ref, b_ref, o_ref):
    o_ref[...] = (
        jnp.dot(x_ref[...], w_ref[...], preferred_element_type=jnp.float32)
        + b_ref[...]
    )


def _spmm1_body(x_ref, w1_ref, b1_ref, a_ref, w_ref, b_ref,
                o_ref, acopy_ref, abf_ref, g_scr):
    @pl.when(pl.program_id(0) == 0)
    def _():
        g_scr[...] = (
            jnp.dot(x_ref[...], w1_ref[...], preferred_element_type=jnp.float32)
            + b1_ref[...]
        ).astype(jnp.bfloat16)

    a = a_ref[...]
    acopy_ref[...] = a
    a_bf = a.astype(jnp.bfloat16)
    abf_ref[...] = a_bf
    y = jnp.dot(a_bf, g_scr[...],
                preferred_element_type=jnp.float32)
    o_ref[...] = (
        jnp.dot(jnp.maximum(y, 0.0), w_ref[...], preferred_element_type=jnp.float32)
        + b_ref[...]
    )


def _spmm23_body(a_ref, g_ref, w_ref, b_ref, h_ref, g3_scr):
    nblk = _N // _BM2
    i = pl.program_id(0)
    base = jax.lax.rem(i, nblk) * _BM2

    @pl.when(i < nblk)
    def _():
        y = jnp.dot(a_ref[...], g_ref[...].astype(jnp.bfloat16),
                    preferred_element_type=jnp.float32)
        g3_scr[pl.ds(base, _BM2), :] = (
            jnp.dot(jnp.maximum(y, 0.0), w_ref[...],
                    preferred_element_type=jnp.float32)
            + b_ref[...]
        ).astype(jnp.bfloat16)

    @pl.when(i >= nblk)
    def _():
        y = jnp.dot(a_ref[...], g3_scr[...],
                    preferred_element_type=jnp.float32)
        denom = jnp.clip(jnp.sum(jnp.abs(y), axis=1, keepdims=True), 1e-12, None)
        h_ref[...] = y / denom


def kernel(x, A, W1, b1, W2, b2, W3, b3):
    f32 = jnp.float32
    params = pltpu.CompilerParams(dimension_semantics=("arbitrary",))

    def specs(bm):
        return dict(
            a=pl.BlockSpec((bm, _N), lambda i: (i, 0)),
            g=pl.BlockSpec((_N, _D), lambda i: (0, 0)),
            w=pl.BlockSpec((_D, _D), lambda i: (0, 0)),
            b=pl.BlockSpec((1, _D), lambda i: (0, 0)),
            o=pl.BlockSpec((bm, _D), lambda i: (i, 0)),
        )

    s1 = specs(_BM1)
    g2, A_copy, A_bf = pl.pallas_call(
        _spmm1_body,
        grid=(_N // _BM1,),
        in_specs=[s1["g"], s1["w"], s1["b"], s1["a"], s1["w"], s1["b"]],
        out_specs=[s1["o"], s1["a"], s1["a"]],
        out_shape=[
            jax.ShapeDtypeStruct((_N, _D), f32),
            jax.ShapeDtypeStruct((_N, _N), f32),
            jax.ShapeDtypeStruct((_N, _N), jnp.bfloat16),
        ],
        scratch_shapes=[pltpu.VMEM((_N, _D), jnp.bfloat16)],
        compiler_params=params,
    )(x, W1, b1[None, :], A, W2, b2[None, :])

    nblk = _N // _BM2
    h = pl.pallas_call(
        _spmm23_body,
        grid=(2 * nblk,),
        in_specs=[
            pl.BlockSpec((_BM2, _N), lambda i: (jax.lax.rem(i, _N // _BM2), 0)),
            pl.BlockSpec((_N, _D), lambda i: (0, 0)),
            pl.BlockSpec((_D, _D), lambda i: (0, 0)),
            pl.BlockSpec((1, _D), lambda i: (0, 0)),
        ],
        out_specs=pl.BlockSpec(
            (_BM2, _D), lambda i: (jnp.maximum(i - _N // _BM2, 0), 0)),
        out_shape=jax.ShapeDtypeStruct((_N, _D), f32),
        scratch_shapes=[pltpu.VMEM((_N, _D), jnp.bfloat16)],
        compiler_params=params,
    )(A_bf, g2, W3, b3[None, :])
    return (h, h, A_copy)


# speedup vs baseline: 1.0698x; 1.0065x over previous
import jax
import jax.numpy as jnp
from jax.experimental import pallas as pl
from jax.experimental.pallas import tpu as pltpu

_N = 10000
_D = 128
_BM1 = 256
_BM2 = 1000


def _transform_body(x_ref, w_ref, b_ref, o_ref):
    o_ref[...] = (
        jnp.dot(x_ref[...], w_ref[...], preferred_element_type=jnp.float32)
        + b_ref[...]
    )


def _spmm1_body(x_ref, w1_ref, b1_ref, a_ref, w_ref, b_ref,
                o_ref, acopy_ref, abf_ref, g_scr):
    @pl.when(pl.program_id(0) == 0)
    def _():
        g_scr[...] = (
            jnp.dot(x_ref[...], w1_ref[...], preferred_element_type=jnp.float32)
            + b1_ref[...]
        ).astype(jnp.bfloat16)

    a = a_ref[...]
    acopy_ref[...] = a
    a_bf = a.astype(jnp.bfloat16)
    abf_ref[...] = a_bf
    y = jnp.dot(a_bf, g_scr[...],
                preferred_element_type=jnp.float32)
    o_ref[...] = (
        jnp.dot(jnp.maximum(y, 0.0), w_ref[...], preferred_element_type=jnp.float32)
        + b_ref[...]
    )


def _spmm23_body(a_ref, g_ref, w_ref, b_ref, h_ref, g3_scr):
    nblk = _N // _BM2
    i = pl.program_id(0)
    base = jax.lax.rem(i, nblk) * _BM2

    @pl.when(i < nblk)
    def _():
        y = jnp.dot(a_ref[...], g_ref[...].astype(jnp.bfloat16),
                    preferred_element_type=jnp.float32)
        g3_scr[pl.ds(base, _BM2), :] = (
            jnp.dot(jnp.maximum(y, 0.0), w_ref[...],
                    preferred_element_type=jnp.float32)
            + b_ref[...]
        ).astype(jnp.bfloat16)

    @pl.when(i >= nblk)
    def _():
        y = jnp.dot(a_ref[...], g3_scr[...],
                    preferred_element_type=jnp.float32)
        denom = jnp.clip(jnp.sum(jnp.abs(y), axis=1, keepdims=True), 1e-12, None)
        h_ref[...] = y / denom


def kernel(x, A, W1, b1, W2, b2, W3, b3):
    f32 = jnp.float32
    params = pltpu.CompilerParams(dimension_semantics=("arbitrary",))

    def specs(bm):
        return dict(
            a=pl.BlockSpec((bm, _N), lambda i: (i, 0)),
            g=pl.BlockSpec((_N, _D), lambda i: (0, 0)),
            w=pl.BlockSpec((_D, _D), lambda i: (0, 0)),
            b=pl.BlockSpec((1, _D), lambda i: (0, 0)),
            o=pl.BlockSpec((bm, _D), lambda i: (i, 0)),
        )

    s1 = specs(_BM1)
    g2, A_copy, A_bf = pl.pallas_call(
        _spmm1_body,
        grid=(pl.cdiv(_N, _BM1),),
        in_specs=[s1["g"], s1["w"], s1["b"], s1["a"], s1["w"], s1["b"]],
        out_specs=[s1["o"], s1["a"], s1["a"]],
        out_shape=[
            jax.ShapeDtypeStruct((_N, _D), f32),
            jax.ShapeDtypeStruct((_N, _N), f32),
            jax.ShapeDtypeStruct((_N, _N), jnp.bfloat16),
        ],
        scratch_shapes=[pltpu.VMEM((_N, _D), jnp.bfloat16)],
        compiler_params=params,
    )(x, W1, b1[None, :], A, W2, b2[None, :])

    nblk = _N // _BM2
    h = pl.pallas_call(
        _spmm23_body,
        grid=(2 * nblk,),
        in_specs=[
            pl.BlockSpec((_BM2, _N), lambda i: (jax.lax.rem(i, _N // _BM2), 0)),
            pl.BlockSpec((_N, _D), lambda i: (0, 0)),
            pl.BlockSpec((_D, _D), lambda i: (0, 0)),
            pl.BlockSpec((1, _D), lambda i: (0, 0)),
        ],
        out_specs=pl.BlockSpec(
            (_BM2, _D), lambda i: (jnp.maximum(i - _N // _BM2, 0), 0)),
        out_shape=jax.ShapeDtypeStruct((_N, _D), f32),
        scratch_shapes=[pltpu.VMEM((_N, _D), jnp.bfloat16)],
        compiler_params=params,
    )(A_bf, g2, W3, b3[None, :])
    return (h, h, A_copy)
